# baseline (speedup 1.0000x reference)
import functools

import jax
import jax.numpy as jnp
from jax import lax
from jax.experimental import pallas as pl
from jax.experimental.pallas import tpu as pltpu
from jax.experimental.pallas import tpu_sc as plsc

_N1 = 1024
_N2 = 1024
_DIM = 32
_EMB = 64
_HEADS = 4
_K = 5
_N = _N1 + _N2
_KP = 128
_NW = 32
_RPW = _N1 // _NW
_NSLOT = 4

_HP = jax.lax.Precision.HIGHEST


def _knn_body(xb_ref, xt_ref, y_ref, idx_ref, yp_ref):
    i = pl.program_id(0)
    rows = yp_ref.shape[0]
    work = jnp.zeros((rows, _N1), jnp.float32)
    for d in range(_DIM):
        diff = xb_ref[:, d:d + 1] - xt_ref[d:d + 1, :]
        work = work + diff * diff
    colk = jax.lax.broadcasted_iota(jnp.int32, (rows, _N1), 1)
    rowk = jax.lax.broadcasted_iota(jnp.int32, (rows, _N1), 0) + i * rows
    work = work + jnp.where(colk == rowk, 1e9, 0.0).astype(jnp.float32)
    acc = jnp.zeros((rows, _N1), jnp.float32)
    for t in range(_K):
        m = jnp.min(work, axis=1, keepdims=True)
        eq = work == m
        idx = jnp.min(jnp.where(eq, colk, jnp.int32(2**30)), axis=1,
                      keepdims=True)
        if t == 0:
            idx_ref[...] = jnp.broadcast_to(idx, (rows, _KP))
        else:
            idx_ref[:, t:t + 1] = idx
        oh = colk == idx
        acc = acc + oh.astype(jnp.float32)
        work = jnp.where(oh, jnp.float32(jnp.inf), work)
    yp_ref[...] = jax.lax.dot(acc, y_ref[...], precision=_HP) * (1.0 / _K)


def _sc_teacher(nn_ref, t_ref, idx_v, pairbuf_v, crow_v, sem_b, *sem_t):
    wid = lax.axis_index("s") * 2 + lax.axis_index("c")
    r0 = wid * _RPW

    pltpu.sync_copy(nn_ref.at[pl.ds(r0, _RPW)], idx_v)

    zeros16 = jnp.zeros((16,), jnp.float32)
    ones16 = jnp.ones((16,), jnp.float32)
    for c in range(_N // 16):
        left = c < (_N1 // 16)
        for r in range(2):
            crow_v[r, pl.ds(c * 16, 16)] = ones16 if left else zeros16
        for r in range(2 * _NSLOT):
            pairbuf_v[r, pl.ds(c * 16, 16)] = zeros16 if left else ones16

    bot = [pltpu.async_copy(crow_v, t_ref.at[pl.ds(_N1 + r0 + 2 * b, 2)],
                            sem_b)
           for b in range(_RPW // 2)]

    lane = lax.iota(jnp.int32, 16)
    lrow = jnp.where(lane >= 8, 1, 0)
    lcol = lane % 8
    top = [None] * _NSLOT
    for p in range(_RPW // 2):
        s = p % _NSLOT
        colidx = plsc.load_gather(idx_v, [2 * p + lrow, lcol])
        rowi = 2 * s + lrow
        if top[s] is not None:
            top[s].wait()
            old = plsc.load_gather(idx_v, [2 * (p - _NSLOT) + lrow, lcol])
            plsc.store_scatter(pairbuf_v, [rowi, old], zeros16)
        plsc.store_scatter(pairbuf_v, [rowi, colidx], ones16)
        top[s] = pltpu.async_copy(
            pairbuf_v.at[pl.ds(2 * s, 2)], t_ref.at[pl.ds(r0 + 2 * p, 2)],
            sem_t[s])
    for h in top:
        h.wait()
    for h in bot:
        h.wait()


def _adj_body(fdl_ref, fdt_ref, w_ref, b_ref, wt_ref, out_ref,
              emb_s, a_s, bb_s):
    i = pl.program_id(0)
    rows = out_ref.shape[0]

    @pl.when(i == 0)
    def _build():
        emb_s[:_N1, :] = (jax.lax.dot(fdl_ref[...], w_ref[...], precision=_HP)
                          + b_ref[...])
        emb_s[_N1:, :] = (jax.lax.dot(fdt_ref[...], w_ref[...], precision=_HP)
                          + b_ref[...])
        emb = emb_s[...]
        for h in range(_HEADS):
            ah = emb * wt_ref[h:h + 1, :]
            bh = emb * wt_ref[_HEADS + h:_HEADS + h + 1, :]
            na = jnp.sqrt(jnp.sum(ah * ah, axis=1, keepdims=True))
            nb = jnp.sqrt(jnp.sum(bh * bh, axis=1, keepdims=True))
            a_s[:, h * _EMB:(h + 1) * _EMB] = (
                ah / jnp.maximum(na, 1e-20)).astype(jnp.bfloat16)
            bb_s[:, h * _EMB:(h + 1) * _EMB] = (
                bh / jnp.maximum(nb, 1e-20)).astype(jnp.bfloat16)

    acc = jnp.zeros((rows, _N), jnp.float32)
    for h in range(_HEADS):
        ah = a_s[pl.ds(i * rows, rows), h * _EMB:(h + 1) * _EMB]
        bh = bb_s[:, h * _EMB:(h + 1) * _EMB]
        dots = jax.lax.dot_general(ah, bh, (((1,), (1,)), ((), ())),
                                   preferred_element_type=jnp.float32)
        acc = acc + jax.nn.sigmoid(dots)
    out_ref[...] = acc * (1.0 / _HEADS)


def kernel(lm_X, lm_Y, tg_X, tg_Y, lm_delay, tg_delay, emb_W, emb_b, w1, w2):
    fd_lm = jnp.concatenate([lm_X, lm_delay[:, None]], axis=1)
    fd_tg = jnp.concatenate([tg_X, tg_delay[:, None]], axis=1)
    wt = jnp.concatenate([w1.T, w2.T], axis=0)

    k_rows = 128
    nn_idx, y_pred = pl.pallas_call(
        _knn_body,
        grid=(_N1 // k_rows,),
        in_specs=[
            pl.BlockSpec((k_rows, _DIM), lambda i: (i, 0)),
            pl.BlockSpec((_DIM, _N1), lambda i: (0, 0)),
            pl.BlockSpec((_N1, 2), lambda i: (0, 0)),
        ],
        out_specs=[
            pl.BlockSpec((k_rows, _KP), lambda i: (i, 0)),
            pl.BlockSpec((k_rows, 2), lambda i: (i, 0)),
        ],
        out_shape=[
            jax.ShapeDtypeStruct((_N1, _KP), jnp.int32),
            jax.ShapeDtypeStruct((_N1, 2), jnp.float32),
        ],
    )(lm_X, lm_X.T, lm_Y)

    sc_fn = pl.kernel(
        _sc_teacher,
        out_type=jax.ShapeDtypeStruct((_N, _N), jnp.float32),
        mesh=plsc.VectorSubcoreMesh(core_axis_name="c", subcore_axis_name="s"),
        compiler_params=pltpu.CompilerParams(needs_layout_passes=False),
        scratch_types=[
            pltpu.VMEM((_RPW, _KP), jnp.int32),
            pltpu.VMEM((2 * _NSLOT, _N), jnp.float32),
            pltpu.VMEM((2, _N), jnp.float32),
            pltpu.SemaphoreType.DMA,
        ] + [pltpu.SemaphoreType.DMA] * _NSLOT,
    )
    teacher = sc_fn(nn_idx)

    wt_dep, _ = jax.lax.optimization_barrier((wt, nn_idx))

    a_rows = 128
    adj = pl.pallas_call(
        _adj_body,
        grid=(_N // a_rows,),
        in_specs=[
            pl.BlockSpec((_N1, _DIM + 1), lambda i: (0, 0)),
            pl.BlockSpec((_N2, _DIM + 1), lambda i: (0, 0)),
            pl.BlockSpec((_DIM + 1, _EMB), lambda i: (0, 0)),
            pl.BlockSpec((1, _EMB), lambda i: (0, 0)),
            pl.BlockSpec((2 * _HEADS, _EMB), lambda i: (0, 0)),
        ],
        out_specs=pl.BlockSpec((a_rows, _N), lambda i: (i, 0)),
        out_shape=jax.ShapeDtypeStruct((_N, _N), jnp.float32),
        scratch_shapes=[
            pltpu.VMEM((_N, _EMB), jnp.float32),
            pltpu.VMEM((_N, _HEADS * _EMB), jnp.bfloat16),
            pltpu.VMEM((_N, _HEADS * _EMB), jnp.bfloat16),
        ],
    )(fd_lm, fd_tg, emb_W, emb_b.reshape(1, _EMB), wt_dep)

    return y_pred, adj, teacher


# speedup vs baseline: 1.1208x; 1.1208x over previous
import functools

import jax
import jax.numpy as jnp
from jax import lax
from jax.experimental import pallas as pl
from jax.experimental.pallas import tpu as pltpu
from jax.experimental.pallas import tpu_sc as plsc

_N1 = 1024
_N2 = 1024
_DIM = 32
_EMB = 64
_HEADS = 4
_K = 5
_N = _N1 + _N2
_KP = 128
_NW = 32
_RPW = _N1 // _NW
_NSLOT = 4

_HP = jax.lax.Precision.HIGHEST


def _knn_body(xb_ref, xt_ref, y_ref, idx_ref, yp_ref):
    i = pl.program_id(0)
    rows = yp_ref.shape[0]
    work = jnp.zeros((rows, _N1), jnp.float32)
    for d in range(_DIM):
        diff = xb_ref[:, d:d + 1] - xt_ref[d:d + 1, :]
        work = work + diff * diff
    colk = jax.lax.broadcasted_iota(jnp.int32, (rows, _N1), 1)
    rowk = jax.lax.broadcasted_iota(jnp.int32, (rows, _N1), 0) + i * rows
    work = work + jnp.where(colk == rowk, 1e9, 0.0).astype(jnp.float32)
    acc = jnp.zeros((rows, _N1), jnp.float32)
    for t in range(_K):
        m = jnp.min(work, axis=1, keepdims=True)
        eq = work == m
        idx = jnp.min(jnp.where(eq, colk, jnp.int32(2**30)), axis=1,
                      keepdims=True)
        if t == 0:
            idx_ref[...] = jnp.broadcast_to(idx, (rows, _KP))
        else:
            idx_ref[:, t:t + 1] = idx
        oh = colk == idx
        acc = acc + oh.astype(jnp.float32)
        work = jnp.where(oh, jnp.float32(jnp.inf), work)
    yp_ref[...] = jax.lax.dot(acc, y_ref[...], precision=_HP) * (1.0 / _K)


def _sc_teacher(nn_ref, t_ref, idx_v, pairbuf_v, crow_v, sem_b, *sem_t):
    wid = lax.axis_index("s") * 2 + lax.axis_index("c")
    r0 = wid * _RPW

    pltpu.sync_copy(nn_ref.at[pl.ds(r0, _RPW)], idx_v)

    zeros16 = jnp.zeros((16,), jnp.float32)
    ones16 = jnp.ones((16,), jnp.float32)
    for c in range(_N // 16):
        left = c < (_N1 // 16)
        for r in range(2):
            crow_v[r, pl.ds(c * 16, 16)] = ones16 if left else zeros16
        for r in range(2 * _NSLOT):
            pairbuf_v[r, pl.ds(c * 16, 16)] = zeros16 if left else ones16

    bot = [pltpu.async_copy(crow_v, t_ref.at[pl.ds(_N1 + r0 + 2 * b, 2)],
                            sem_b)
           for b in range(_RPW // 2)]

    lane = lax.iota(jnp.int32, 16)
    lrow = jnp.where(lane >= 8, 1, 0)
    lcol = lane % 8
    top = [None] * _NSLOT
    for p in range(_RPW // 2):
        s = p % _NSLOT
        colidx = plsc.load_gather(idx_v, [2 * p + lrow, lcol])
        rowi = 2 * s + lrow
        if top[s] is not None:
            top[s].wait()
            old = plsc.load_gather(idx_v, [2 * (p - _NSLOT) + lrow, lcol])
            plsc.store_scatter(pairbuf_v, [rowi, old], zeros16)
        plsc.store_scatter(pairbuf_v, [rowi, colidx], ones16)
        top[s] = pltpu.async_copy(
            pairbuf_v.at[pl.ds(2 * s, 2)], t_ref.at[pl.ds(r0 + 2 * p, 2)],
            sem_t[s])
    for h in top:
        h.wait()
    for h in bot:
        h.wait()


def _adj_body(x_ref, dly_ref, w_ref, b_ref, wt_ref, out_ref,
              emb_s, a_s, bb_s):
    i = pl.program_id(0)
    rows = out_ref.shape[0]

    @pl.when(i == 0)
    def _build():
        dcol = jax.lax.dot_general(
            dly_ref[...], w_ref[_DIM:_DIM + 1, :], (((0,), (0,)), ((), ())),
            precision=_HP)
        emb = (jax.lax.dot(x_ref[...], w_ref[:_DIM, :], precision=_HP)
               + dcol + b_ref[...])
        emb_s[...] = emb
        for h in range(_HEADS):
            ah = emb * wt_ref[h:h + 1, :]
            bh = emb * wt_ref[_HEADS + h:_HEADS + h + 1, :]
            na = jnp.sqrt(jnp.sum(ah * ah, axis=1, keepdims=True))
            nb = jnp.sqrt(jnp.sum(bh * bh, axis=1, keepdims=True))
            a_s[:, h * _EMB:(h + 1) * _EMB] = (
                ah / jnp.maximum(na, 1e-20)).astype(jnp.bfloat16)
            bb_s[:, h * _EMB:(h + 1) * _EMB] = (
                bh / jnp.maximum(nb, 1e-20)).astype(jnp.bfloat16)

    acc = jnp.zeros((rows, _N), jnp.float32)
    for h in range(_HEADS):
        ah = a_s[pl.ds(i * rows, rows), h * _EMB:(h + 1) * _EMB]
        bh = bb_s[:, h * _EMB:(h + 1) * _EMB]
        dots = jax.lax.dot_general(ah, bh, (((1,), (1,)), ((), ())),
                                   preferred_element_type=jnp.float32)
        acc = acc + jnp.tanh(dots * 0.5)
    out_ref[...] = acc * (0.5 / _HEADS) + 0.5


def kernel(lm_X, lm_Y, tg_X, tg_Y, lm_delay, tg_delay, emb_W, emb_b, w1, w2):
    x_all = jnp.concatenate([lm_X, tg_X], axis=0)
    dly_row = jnp.concatenate([lm_delay, tg_delay])[None, :]
    wt = jnp.concatenate([w1.T, w2.T], axis=0)

    k_rows = 128
    nn_idx, y_pred = pl.pallas_call(
        _knn_body,
        grid=(_N1 // k_rows,),
        in_specs=[
            pl.BlockSpec((k_rows, _DIM), lambda i: (i, 0)),
            pl.BlockSpec((_DIM, _N1), lambda i: (0, 0)),
            pl.BlockSpec((_N1, 2), lambda i: (0, 0)),
        ],
        out_specs=[
            pl.BlockSpec((k_rows, _KP), lambda i: (i, 0)),
            pl.BlockSpec((k_rows, 2), lambda i: (i, 0)),
        ],
        out_shape=[
            jax.ShapeDtypeStruct((_N1, _KP), jnp.int32),
            jax.ShapeDtypeStruct((_N1, 2), jnp.float32),
        ],
    )(lm_X, lm_X.T, lm_Y)

    sc_fn = pl.kernel(
        _sc_teacher,
        out_type=jax.ShapeDtypeStruct((_N, _N), jnp.float32),
        mesh=plsc.VectorSubcoreMesh(core_axis_name="c", subcore_axis_name="s"),
        compiler_params=pltpu.CompilerParams(needs_layout_passes=False),
        scratch_types=[
            pltpu.VMEM((_RPW, _KP), jnp.int32),
            pltpu.VMEM((2 * _NSLOT, _N), jnp.float32),
            pltpu.VMEM((2, _N), jnp.float32),
            pltpu.SemaphoreType.DMA,
        ] + [pltpu.SemaphoreType.DMA] * _NSLOT,
    )
    teacher = sc_fn(nn_idx)

    wt_dep, _ = jax.lax.optimization_barrier((wt, nn_idx))

    a_rows = 256
    adj = pl.pallas_call(
        _adj_body,
        grid=(_N // a_rows,),
        in_specs=[
            pl.BlockSpec((_N, _DIM), lambda i: (0, 0)),
            pl.BlockSpec((1, _N), lambda i: (0, 0)),
            pl.BlockSpec((_DIM + 1, _EMB), lambda i: (0, 0)),
            pl.BlockSpec((1, _EMB), lambda i: (0, 0)),
            pl.BlockSpec((2 * _HEADS, _EMB), lambda i: (0, 0)),
        ],
        out_specs=pl.BlockSpec((a_rows, _N), lambda i: (i, 0)),
        out_shape=jax.ShapeDtypeStruct((_N, _N), jnp.float32),
        scratch_shapes=[
            pltpu.VMEM((_N, _EMB), jnp.float32),
            pltpu.VMEM((_N, _HEADS * _EMB), jnp.bfloat16),
            pltpu.VMEM((_N, _HEADS * _EMB), jnp.bfloat16),
        ],
    )(x_all, dly_row, emb_W, emb_b.reshape(1, _EMB), wt_dep)

    return y_pred, adj, teacher


# speedup vs baseline: 1.1265x; 1.0050x over previous
import functools

import jax
import jax.numpy as jnp
from jax import lax
from jax.experimental import pallas as pl
from jax.experimental.pallas import tpu as pltpu
from jax.experimental.pallas import tpu_sc as plsc

_N1 = 1024
_N2 = 1024
_DIM = 32
_EMB = 64
_HEADS = 4
_K = 5
_N = _N1 + _N2
_KP = 128
_NW = 32
_RPW = _N1 // _NW
_NSLOT = 4

_HP = jax.lax.Precision.HIGHEST


def _knn_body(xb_ref, xt_ref, y_ref, idx_ref, yp_ref):
    i = pl.program_id(0)
    rows = yp_ref.shape[0]
    work = jnp.zeros((rows, _N1), jnp.float32)
    for d in range(_DIM):
        diff = xb_ref[:, d:d + 1] - xt_ref[d:d + 1, :]
        work = work + diff * diff
    colk = jax.lax.broadcasted_iota(jnp.int32, (rows, _N1), 1)
    rowk = jax.lax.broadcasted_iota(jnp.int32, (rows, _N1), 0) + i * rows
    work = work + jnp.where(colk == rowk, 1e9, 0.0).astype(jnp.float32)
    acc = jnp.zeros((rows, _N1), jnp.float32)
    for t in range(_K):
        m = jnp.min(work, axis=1, keepdims=True)
        eq = work == m
        idx = jnp.min(jnp.where(eq, colk, jnp.int32(2**30)), axis=1,
                      keepdims=True)
        if t == 0:
            idx_ref[...] = jnp.broadcast_to(idx, (rows, _KP))
        else:
            idx_ref[:, t:t + 1] = idx
        oh = colk == idx
        acc = acc + oh.astype(jnp.float32)
        work = jnp.where(oh, jnp.float32(jnp.inf), work)
    yp_ref[...] = jax.lax.dot(acc, y_ref[...], precision=_HP) * (1.0 / _K)


def _sc_teacher(nn_ref, t_ref, idx_v, pairbuf_v, crow_v, sem_b, *sem_t):
    wid = lax.axis_index("s") * 2 + lax.axis_index("c")
    r0 = wid * _RPW

    pltpu.sync_copy(nn_ref.at[pl.ds(r0, _RPW)], idx_v)

    zeros16 = jnp.zeros((16,), jnp.float32)
    ones16 = jnp.ones((16,), jnp.float32)
    for c in range(_N // 16):
        left = c < (_N1 // 16)
        for r in range(2):
            crow_v[r, pl.ds(c * 16, 16)] = ones16 if left else zeros16
        for r in range(2 * _NSLOT):
            pairbuf_v[r, pl.ds(c * 16, 16)] = zeros16 if left else ones16

    bot = [pltpu.async_copy(crow_v, t_ref.at[pl.ds(_N1 + r0 + 2 * b, 2)],
                            sem_b)
           for b in range(_RPW // 2)]

    lane = lax.iota(jnp.int32, 16)
    lrow = jnp.where(lane >= 8, 1, 0)
    lcol = lane % 8
    top = [None] * _NSLOT
    for p in range(_RPW // 2):
        s = p % _NSLOT
        colidx = plsc.load_gather(idx_v, [2 * p + lrow, lcol])
        rowi = 2 * s + lrow
        if top[s] is not None:
            top[s].wait()
            old = plsc.load_gather(idx_v, [2 * (p - _NSLOT) + lrow, lcol])
            plsc.store_scatter(pairbuf_v, [rowi, old], zeros16)
        plsc.store_scatter(pairbuf_v, [rowi, colidx], ones16)
        top[s] = pltpu.async_copy(
            pairbuf_v.at[pl.ds(2 * s, 2)], t_ref.at[pl.ds(r0 + 2 * p, 2)],
            sem_t[s])
    for h in top:
        h.wait()
    for h in bot:
        h.wait()


def _adj_body(xl_ref, xt_ref, dly_ref, w_ref, wt_ref, out_ref,
              emb_s, a_s, bb_s):
    i = pl.program_id(0)
    rows = out_ref.shape[0]

    @pl.when(i == 0)
    def _build():
        dcol = jax.lax.dot_general(
            dly_ref[...], w_ref[_DIM:_DIM + 1, :], (((0,), (0,)), ((), ())),
            precision=_HP)
        emb_s[:_N1, :] = jax.lax.dot(xl_ref[...], w_ref[:_DIM, :],
                                     precision=_HP)
        emb_s[_N1:, :] = jax.lax.dot(xt_ref[...], w_ref[:_DIM, :],
                                     precision=_HP)
        emb = emb_s[...] + dcol + wt_ref[2 * _HEADS:2 * _HEADS + 1, :]
        emb_s[...] = emb
        for h in range(_HEADS):
            ah = emb * wt_ref[h:h + 1, :]
            bh = emb * wt_ref[_HEADS + h:_HEADS + h + 1, :]
            na = jnp.sqrt(jnp.sum(ah * ah, axis=1, keepdims=True))
            nb = jnp.sqrt(jnp.sum(bh * bh, axis=1, keepdims=True))
            a_s[:, h * _EMB:(h + 1) * _EMB] = (
                ah / jnp.maximum(na, 1e-20)).astype(jnp.bfloat16)
            bb_s[:, h * _EMB:(h + 1) * _EMB] = (
                bh / jnp.maximum(nb, 1e-20)).astype(jnp.bfloat16)

    acc = jnp.zeros((rows, _N), jnp.float32)
    for h in range(_HEADS):
        ah = a_s[pl.ds(i * rows, rows), h * _EMB:(h + 1) * _EMB]
        bh = bb_s[:, h * _EMB:(h + 1) * _EMB]
        dots = jax.lax.dot_general(ah, bh, (((1,), (1,)), ((), ())),
                                   preferred_element_type=jnp.float32)
        acc = acc + jnp.tanh(dots * 0.5)
    out_ref[...] = acc * (0.5 / _HEADS) + 0.5


def kernel(lm_X, lm_Y, tg_X, tg_Y, lm_delay, tg_delay, emb_W, emb_b, w1, w2):
    dly_row = jnp.concatenate([lm_delay, tg_delay])[None, :]
    wt = jnp.concatenate([w1.T, w2.T, emb_b[None, :]], axis=0)

    k_rows = 128
    nn_idx, y_pred = pl.pallas_call(
        _knn_body,
        grid=(_N1 // k_rows,),
        in_specs=[
            pl.BlockSpec((k_rows, _DIM), lambda i: (i, 0)),
            pl.BlockSpec((_DIM, _N1), lambda i: (0, 0)),
            pl.BlockSpec((_N1, 2), lambda i: (0, 0)),
        ],
        out_specs=[
            pl.BlockSpec((k_rows, _KP), lambda i: (i, 0)),
            pl.BlockSpec((k_rows, 2), lambda i: (i, 0)),
        ],
        out_shape=[
            jax.ShapeDtypeStruct((_N1, _KP), jnp.int32),
            jax.ShapeDtypeStruct((_N1, 2), jnp.float32),
        ],
    )(lm_X, lm_X.T, lm_Y)

    sc_fn = pl.kernel(
        _sc_teacher,
        out_type=jax.ShapeDtypeStruct((_N, _N), jnp.float32),
        mesh=plsc.VectorSubcoreMesh(core_axis_name="c", subcore_axis_name="s"),
        compiler_params=pltpu.CompilerParams(needs_layout_passes=False),
        scratch_types=[
            pltpu.VMEM((_RPW, _KP), jnp.int32),
            pltpu.VMEM((2 * _NSLOT, _N), jnp.float32),
            pltpu.VMEM((2, _N), jnp.float32),
            pltpu.SemaphoreType.DMA,
        ] + [pltpu.SemaphoreType.DMA] * _NSLOT,
    )
    teacher = sc_fn(nn_idx)

    wt_dep, _ = jax.lax.optimization_barrier((wt, nn_idx))

    a_rows = 256
    adj = pl.pallas_call(
        _adj_body,
        grid=(_N // a_rows,),
        in_specs=[
            pl.BlockSpec((_N1, _DIM), lambda i: (0, 0)),
            pl.BlockSpec((_N2, _DIM), lambda i: (0, 0)),
            pl.BlockSpec((1, _N), lambda i: (0, 0)),
            pl.BlockSpec((_DIM + 1, _EMB), lambda i: (0, 0)),
            pl.BlockSpec((2 * _HEADS + 1, _EMB), lambda i: (0, 0)),
        ],
        out_specs=pl.BlockSpec((a_rows, _N), lambda i: (i, 0)),
        out_shape=jax.ShapeDtypeStruct((_N, _N), jnp.float32),
        scratch_shapes=[
            pltpu.VMEM((_N, _EMB), jnp.float32),
            pltpu.VMEM((_N, _HEADS * _EMB), jnp.bfloat16),
            pltpu.VMEM((_N, _HEADS * _EMB), jnp.bfloat16),
        ],
    )(lm_X, tg_X, dly_row, emb_W, wt_dep)

    return y_pred, adj, teacher


# speedup vs baseline: 1.1858x; 1.0527x over previous
import functools

import jax
import jax.numpy as jnp
from jax import lax
from jax.experimental import pallas as pl
from jax.experimental.pallas import tpu as pltpu
from jax.experimental.pallas import tpu_sc as plsc

_N1 = 1024
_N2 = 1024
_DIM = 32
_EMB = 64
_HEADS = 4
_K = 5
_N = _N1 + _N2
_KP = 128
_NW = 32
_RPW = _N1 // _NW
_NSLOT = 4

_HP = jax.lax.Precision.HIGHEST


def _knn_body(xb_ref, xt_ref, y_ref, idx_ref, yp_ref):
    i = pl.program_id(0)
    rows = yp_ref.shape[0]
    work = jnp.zeros((rows, _N1), jnp.float32)
    for d in range(_DIM):
        diff = xb_ref[:, d:d + 1] - xt_ref[d:d + 1, :]
        work = work + diff * diff
    colk = jax.lax.broadcasted_iota(jnp.int32, (rows, _N1), 1)
    rowk = jax.lax.broadcasted_iota(jnp.int32, (rows, _N1), 0) + i * rows
    work = work + jnp.where(colk == rowk, 1e9, 0.0).astype(jnp.float32)
    acc = jnp.zeros((rows, _N1), jnp.float32)
    for t in range(_K):
        m = jnp.min(work, axis=1, keepdims=True)
        eq = work == m
        idx = jnp.min(jnp.where(eq, colk, jnp.int32(2**30)), axis=1,
                      keepdims=True)
        if t == 0:
            idx_ref[...] = jnp.broadcast_to(idx, (rows, _KP))
        else:
            idx_ref[:, t:t + 1] = idx
        oh = colk == idx
        acc = acc + oh.astype(jnp.float32)
        work = jnp.where(oh, jnp.float32(jnp.inf), work)
    yp_ref[...] = jax.lax.dot(acc, y_ref[...], precision=_HP) * (1.0 / _K)


def _sc_teacher(nn_ref, t_ref, idx_v, pairbuf_v, crow_v, sem_b, *sem_t):
    wid = lax.axis_index("s") * 2 + lax.axis_index("c")
    r0 = wid * _RPW

    pltpu.sync_copy(nn_ref.at[pl.ds(r0, _RPW)], idx_v)

    zeros16 = jnp.zeros((16,), jnp.float32)
    ones16 = jnp.ones((16,), jnp.float32)
    for c in range(_N // 16):
        left = c < (_N1 // 16)
        for r in range(2):
            crow_v[r, pl.ds(c * 16, 16)] = ones16 if left else zeros16
        for r in range(2 * _NSLOT):
            pairbuf_v[r, pl.ds(c * 16, 16)] = zeros16 if left else ones16

    bot = [pltpu.async_copy(crow_v, t_ref.at[pl.ds(_N1 + r0 + 2 * b, 2)],
                            sem_b)
           for b in range(_RPW // 2)]

    lane = lax.iota(jnp.int32, 16)
    lrow = jnp.where(lane >= 8, 1, 0)
    lcol = lane % 8
    top = [None] * _NSLOT
    for p in range(_RPW // 2):
        s = p % _NSLOT
        colidx = plsc.load_gather(idx_v, [2 * p + lrow, lcol])
        rowi = 2 * s + lrow
        if top[s] is not None:
            top[s].wait()
            old = plsc.load_gather(idx_v, [2 * (p - _NSLOT) + lrow, lcol])
            plsc.store_scatter(pairbuf_v, [rowi, old], zeros16)
        plsc.store_scatter(pairbuf_v, [rowi, colidx], ones16)
        top[s] = pltpu.async_copy(
            pairbuf_v.at[pl.ds(2 * s, 2)], t_ref.at[pl.ds(r0 + 2 * p, 2)],
            sem_t[s])
    for h in top:
        h.wait()
    for h in bot:
        h.wait()


def _adj_body(xl_ref, xt_ref, dly_ref, w_ref, wt_ref, out_ref,
              emb_s, a_s, bb_s):
    i = pl.program_id(0)
    rows = out_ref.shape[0]

    @pl.when(i == 0)
    def _build():
        dcol = jax.lax.dot_general(
            dly_ref[...], w_ref[_DIM:_DIM + 1, :], (((0,), (0,)), ((), ())),
            precision=_HP)
        emb_s[:_N1, :] = jax.lax.dot(xl_ref[...], w_ref[:_DIM, :],
                                     precision=_HP)
        emb_s[_N1:, :] = jax.lax.dot(xt_ref[...], w_ref[:_DIM, :],
                                     precision=_HP)
        emb = emb_s[...] + dcol + wt_ref[2 * _HEADS:2 * _HEADS + 1, :]
        emb_s[...] = emb
        for h in range(_HEADS):
            ah = emb * wt_ref[h:h + 1, :]
            bh = emb * wt_ref[_HEADS + h:_HEADS + h + 1, :]
            na = jnp.sqrt(jnp.sum(ah * ah, axis=1, keepdims=True))
            nb = jnp.sqrt(jnp.sum(bh * bh, axis=1, keepdims=True))
            a_s[:, h * _EMB:(h + 1) * _EMB] = (
                ah / jnp.maximum(na, 1e-20)).astype(jnp.bfloat16)
            bb_s[:, h * _EMB:(h + 1) * _EMB] = (
                bh / jnp.maximum(nb, 1e-20)).astype(jnp.bfloat16)

    acc = jnp.zeros((rows, _N), jnp.float32)
    for h in range(_HEADS):
        ah = a_s[pl.ds(i * rows, rows), h * _EMB:(h + 1) * _EMB]
        bh = bb_s[:, h * _EMB:(h + 1) * _EMB]
        dots = jax.lax.dot_general(ah, bh, (((1,), (1,)), ((), ())),
                                   preferred_element_type=jnp.float32)
        acc = acc + jnp.tanh(dots * 0.5)
    out_ref[...] = acc * (0.5 / _HEADS) + 0.5


def kernel(lm_X, lm_Y, tg_X, tg_Y, lm_delay, tg_delay, emb_W, emb_b, w1, w2):
    dly_row = jnp.concatenate([lm_delay, tg_delay])[None, :]
    wt = jnp.concatenate([w1.T, w2.T, emb_b[None, :]], axis=0)

    k_rows = 256
    nn_idx, y_pred = pl.pallas_call(
        _knn_body,
        grid=(_N1 // k_rows,),
        in_specs=[
            pl.BlockSpec((k_rows, _DIM), lambda i: (i, 0)),
            pl.BlockSpec((_DIM, _N1), lambda i: (0, 0)),
            pl.BlockSpec((_N1, 2), lambda i: (0, 0)),
        ],
        out_specs=[
            pl.BlockSpec((k_rows, _KP), lambda i: (i, 0)),
            pl.BlockSpec((k_rows, 2), lambda i: (i, 0)),
        ],
        out_shape=[
            jax.ShapeDtypeStruct((_N1, _KP), jnp.int32),
            jax.ShapeDtypeStruct((_N1, 2), jnp.float32),
        ],
    )(lm_X, lm_X.T, lm_Y)

    sc_fn = pl.kernel(
        _sc_teacher,
        out_type=jax.ShapeDtypeStruct((_N, _N), jnp.float32),
        mesh=plsc.VectorSubcoreMesh(core_axis_name="c", subcore_axis_name="s"),
        compiler_params=pltpu.CompilerParams(needs_layout_passes=False),
        scratch_types=[
            pltpu.VMEM((_RPW, _KP), jnp.int32),
            pltpu.VMEM((2 * _NSLOT, _N), jnp.float32),
            pltpu.VMEM((2, _N), jnp.float32),
            pltpu.SemaphoreType.DMA,
        ] + [pltpu.SemaphoreType.DMA] * _NSLOT,
    )
    teacher = sc_fn(nn_idx)

    wt_dep, _ = jax.lax.optimization_barrier((wt, nn_idx))

    a_rows = 256
    adj = pl.pallas_call(
        _adj_body,
        grid=(_N // a_rows,),
        in_specs=[
            pl.BlockSpec((_N1, _DIM), lambda i: (0, 0)),
            pl.BlockSpec((_N2, _DIM), lambda i: (0, 0)),
            pl.BlockSpec((1, _N), lambda i: (0, 0)),
            pl.BlockSpec((_DIM + 1, _EMB), lambda i: (0, 0)),
            pl.BlockSpec((2 * _HEADS + 1, _EMB), lambda i: (0, 0)),
        ],
        out_specs=pl.BlockSpec((a_rows, _N), lambda i: (i, 0)),
        out_shape=jax.ShapeDtypeStruct((_N, _N), jnp.float32),
        scratch_shapes=[
            pltpu.VMEM((_N, _EMB), jnp.float32),
            pltpu.VMEM((_N, _HEADS * _EMB), jnp.bfloat16),
            pltpu.VMEM((_N, _HEADS * _EMB), jnp.bfloat16),
        ],
    )(lm_X, tg_X, dly_row, emb_W, wt_dep)

    return y_pred, adj, teacher


# speedup vs baseline: 1.1947x; 1.0075x over previous
import functools

import jax
import jax.numpy as jnp
from jax import lax
from jax.experimental import pallas as pl
from jax.experimental.pallas import tpu as pltpu
from jax.experimental.pallas import tpu_sc as plsc

_N1 = 1024
_N2 = 1024
_DIM = 32
_EMB = 64
_HEADS = 4
_K = 5
_N = _N1 + _N2
_KP = 128
_NW = 32
_RPW = _N1 // _NW
_NSLOT = 4

_HP = jax.lax.Precision.HIGHEST


def _knn_body(xb_ref, xt_ref, y_ref, idx_ref, yp_ref):
    i = pl.program_id(0)
    rows = yp_ref.shape[0]
    work = jnp.zeros((rows, _N1), jnp.float32)
    for d in range(_DIM):
        diff = xb_ref[:, d:d + 1] - xt_ref[d:d + 1, :]
        work = work + diff * diff
    colk = jax.lax.broadcasted_iota(jnp.int32, (rows, _N1), 1)
    rowk = jax.lax.broadcasted_iota(jnp.int32, (rows, _N1), 0) + i * rows
    work = work + jnp.where(colk == rowk, 1e9, 0.0).astype(jnp.float32)
    acc = jnp.zeros((rows, _N1), jnp.float32)
    for t in range(_K):
        m = jnp.min(work, axis=1, keepdims=True)
        eq = work == m
        idx = jnp.min(jnp.where(eq, colk, jnp.int32(2**30)), axis=1,
                      keepdims=True)
        if t == 0:
            idx_ref[...] = jnp.broadcast_to(idx, (rows, _KP))
        else:
            idx_ref[:, t:t + 1] = idx
        oh = colk == idx
        acc = acc + oh.astype(jnp.float32)
        work = jnp.where(oh, jnp.float32(jnp.inf), work)
    yp_ref[...] = jax.lax.dot(acc, y_ref[...], precision=_HP) * (1.0 / _K)


def _sc_teacher(nn_ref, t_ref, idx_v, pairbuf_v, crow_v, sem_b, *sem_t):
    wid = lax.axis_index("s") * 2 + lax.axis_index("c")
    r0 = wid * _RPW

    pltpu.sync_copy(nn_ref.at[pl.ds(r0, _RPW)], idx_v)

    zeros16 = jnp.zeros((16,), jnp.float32)
    ones16 = jnp.ones((16,), jnp.float32)
    for c in range(_N // 16):
        left = c < (_N1 // 16)
        for r in range(2):
            crow_v[r, pl.ds(c * 16, 16)] = ones16 if left else zeros16
        for r in range(2 * _NSLOT):
            pairbuf_v[r, pl.ds(c * 16, 16)] = zeros16 if left else ones16

    bot = [pltpu.async_copy(crow_v, t_ref.at[pl.ds(_N1 + r0 + 2 * b, 2)],
                            sem_b)
           for b in range(_RPW // 2)]

    lane = lax.iota(jnp.int32, 16)
    lrow = jnp.where(lane >= 8, 1, 0)
    lcol = lane % 8
    top = [None] * _NSLOT
    for p in range(_RPW // 2):
        s = p % _NSLOT
        colidx = plsc.load_gather(idx_v, [2 * p + lrow, lcol])
        rowi = 2 * s + lrow
        if top[s] is not None:
            top[s].wait()
            old = plsc.load_gather(idx_v, [2 * (p - _NSLOT) + lrow, lcol])
            plsc.store_scatter(pairbuf_v, [rowi, old], zeros16)
        plsc.store_scatter(pairbuf_v, [rowi, colidx], ones16)
        top[s] = pltpu.async_copy(
            pairbuf_v.at[pl.ds(2 * s, 2)], t_ref.at[pl.ds(r0 + 2 * p, 2)],
            sem_t[s])
    for h in top:
        h.wait()
    for h in bot:
        h.wait()


def _adj_body(xl_ref, xt_ref, dly_ref, w_ref, wt_ref, out_ref,
              emb_s, a_s, bb_s):
    i = pl.program_id(0)
    rows = out_ref.shape[0]

    @pl.when(i == 0)
    def _build():
        dcol = jax.lax.dot_general(
            dly_ref[...], w_ref[_DIM:_DIM + 1, :], (((0,), (0,)), ((), ())),
            precision=_HP)
        emb_s[:_N1, :] = jax.lax.dot(xl_ref[...], w_ref[:_DIM, :],
                                     precision=_HP)
        emb_s[_N1:, :] = jax.lax.dot(xt_ref[...], w_ref[:_DIM, :],
                                     precision=_HP)
        emb = emb_s[...] + dcol + wt_ref[2 * _HEADS:2 * _HEADS + 1, :]
        emb_s[...] = emb
        for h in range(_HEADS):
            ah = emb * wt_ref[h:h + 1, :]
            bh = emb * wt_ref[_HEADS + h:_HEADS + h + 1, :]
            na = jnp.sqrt(jnp.sum(ah * ah, axis=1, keepdims=True))
            nb = jnp.sqrt(jnp.sum(bh * bh, axis=1, keepdims=True))
            a_s[:, h * _EMB:(h + 1) * _EMB] = (
                ah / jnp.maximum(na, 1e-20)).astype(jnp.bfloat16)
            bb_s[:, h * _EMB:(h + 1) * _EMB] = (
                bh / jnp.maximum(nb, 1e-20)).astype(jnp.bfloat16)

    acc = jnp.zeros((rows, _N), jnp.float32)
    for h in range(_HEADS):
        ah = a_s[pl.ds(i * rows, rows), h * _EMB:(h + 1) * _EMB]
        bh = bb_s[:, h * _EMB:(h + 1) * _EMB]
        dots = jax.lax.dot_general(ah, bh, (((1,), (1,)), ((), ())),
                                   preferred_element_type=jnp.float32)
        acc = acc + jnp.tanh(dots * 0.5)
    out_ref[...] = acc * (0.5 / _HEADS) + 0.5


def kernel(lm_X, lm_Y, tg_X, tg_Y, lm_delay, tg_delay, emb_W, emb_b, w1, w2):
    dly_row = jnp.concatenate([lm_delay, tg_delay])[None, :]
    wt = jnp.concatenate([w1.T, w2.T, emb_b[None, :]], axis=0)

    k_rows = 512
    nn_idx, y_pred = pl.pallas_call(
        _knn_body,
        grid=(_N1 // k_rows,),
        in_specs=[
            pl.BlockSpec((k_rows, _DIM), lambda i: (i, 0)),
            pl.BlockSpec((_DIM, _N1), lambda i: (0, 0)),
            pl.BlockSpec((_N1, 2), lambda i: (0, 0)),
        ],
        out_specs=[
            pl.BlockSpec((k_rows, _KP), lambda i: (i, 0)),
            pl.BlockSpec((k_rows, 2), lambda i: (i, 0)),
        ],
        out_shape=[
            jax.ShapeDtypeStruct((_N1, _KP), jnp.int32),
            jax.ShapeDtypeStruct((_N1, 2), jnp.float32),
        ],
    )(lm_X, lm_X.T, lm_Y)

    sc_fn = pl.kernel(
        _sc_teacher,
        out_type=jax.ShapeDtypeStruct((_N, _N), jnp.float32),
        mesh=plsc.VectorSubcoreMesh(core_axis_name="c", subcore_axis_name="s"),
        compiler_params=pltpu.CompilerParams(needs_layout_passes=False),
        scratch_types=[
            pltpu.VMEM((_RPW, _KP), jnp.int32),
            pltpu.VMEM((2 * _NSLOT, _N), jnp.float32),
            pltpu.VMEM((2, _N), jnp.float32),
            pltpu.SemaphoreType.DMA,
        ] + [pltpu.SemaphoreType.DMA] * _NSLOT,
    )
    teacher = sc_fn(nn_idx)

    wt_dep, _ = jax.lax.optimization_barrier((wt, nn_idx))

    a_rows = 256
    adj = pl.pallas_call(
        _adj_body,
        grid=(_N // a_rows,),
        in_specs=[
            pl.BlockSpec((_N1, _DIM), lambda i: (0, 0)),
            pl.BlockSpec((_N2, _DIM), lambda i: (0, 0)),
            pl.BlockSpec((1, _N), lambda i: (0, 0)),
            pl.BlockSpec((_DIM + 1, _EMB), lambda i: (0, 0)),
            pl.BlockSpec((2 * _HEADS + 1, _EMB), lambda i: (0, 0)),
        ],
        out_specs=pl.BlockSpec((a_rows, _N), lambda i: (i, 0)),
        out_shape=jax.ShapeDtypeStruct((_N, _N), jnp.float32),
        scratch_shapes=[
            pltpu.VMEM((_N, _EMB), jnp.float32),
            pltpu.VMEM((_N, _HEADS * _EMB), jnp.bfloat16),
            pltpu.VMEM((_N, _HEADS * _EMB), jnp.bfloat16),
        ],
    )(lm_X, tg_X, dly_row, emb_W, wt_dep)

    return y_pred, adj, teacher


# speedup vs baseline: 1.2035x; 1.0074x over previous
import functools

import jax
import jax.numpy as jnp
from jax import lax
from jax.experimental import pallas as pl
from jax.experimental.pallas import tpu as pltpu
from jax.experimental.pallas import tpu_sc as plsc

_N1 = 1024
_N2 = 1024
_DIM = 32
_EMB = 64
_HEADS = 4
_K = 5
_N = _N1 + _N2
_KP = 128
_NW = 32
_RPW = _N1 // _NW
_NSLOT = 4

_HP = jax.lax.Precision.HIGHEST


def _knn_body(xb_ref, xt_ref, y_ref, idx_ref, yp_ref):
    i = pl.program_id(0)
    rows = yp_ref.shape[0]
    work = jnp.zeros((rows, _N1), jnp.float32)
    for d in range(_DIM):
        diff = xb_ref[:, d:d + 1] - xt_ref[d:d + 1, :]
        work = work + diff * diff
    colk = jax.lax.broadcasted_iota(jnp.int32, (rows, _N1), 1)
    rowk = jax.lax.broadcasted_iota(jnp.int32, (rows, _N1), 0) + i * rows
    work = work + jnp.where(colk == rowk, 1e9, 0.0).astype(jnp.float32)
    acc = jnp.zeros((rows, _N1), jnp.float32)
    for t in range(_K):
        m = jnp.min(work, axis=1, keepdims=True)
        eq = work == m
        idx = jnp.min(jnp.where(eq, colk, jnp.int32(2**30)), axis=1,
                      keepdims=True)
        if t == 0:
            idx_ref[...] = jnp.broadcast_to(idx, (rows, _KP))
        else:
            idx_ref[:, t:t + 1] = idx
        oh = colk == idx
        acc = acc + oh.astype(jnp.float32)
        work = jnp.where(oh, jnp.float32(jnp.inf), work)
    yp_ref[...] = jax.lax.dot(acc, y_ref[...], precision=_HP) * (1.0 / _K)


def _sc_teacher(nn_ref, t_ref, idx_v, pairbuf_v, crow_v, sem_b, *sem_t):
    wid = lax.axis_index("s") * 2 + lax.axis_index("c")
    r0 = wid * _RPW

    pltpu.sync_copy(nn_ref.at[pl.ds(r0, _RPW)], idx_v)

    zeros16 = jnp.zeros((16,), jnp.float32)
    ones16 = jnp.ones((16,), jnp.float32)
    for c in range(_N // 16):
        left = c < (_N1 // 16)
        for r in range(2):
            crow_v[r, pl.ds(c * 16, 16)] = ones16 if left else zeros16
        for r in range(2 * _NSLOT):
            pairbuf_v[r, pl.ds(c * 16, 16)] = zeros16 if left else ones16

    bot = [pltpu.async_copy(crow_v, t_ref.at[pl.ds(_N1 + r0 + 2 * b, 2)],
                            sem_b)
           for b in range(_RPW // 2)]

    lane = lax.iota(jnp.int32, 16)
    lrow = jnp.where(lane >= 8, 1, 0)
    lcol = lane % 8
    top = [None] * _NSLOT
    for p in range(_RPW // 2):
        s = p % _NSLOT
        colidx = plsc.load_gather(idx_v, [2 * p + lrow, lcol])
        rowi = 2 * s + lrow
        if top[s] is not None:
            top[s].wait()
            old = plsc.load_gather(idx_v, [2 * (p - _NSLOT) + lrow, lcol])
            plsc.store_scatter(pairbuf_v, [rowi, old], zeros16)
        plsc.store_scatter(pairbuf_v, [rowi, colidx], ones16)
        top[s] = pltpu.async_copy(
            pairbuf_v.at[pl.ds(2 * s, 2)], t_ref.at[pl.ds(r0 + 2 * p, 2)],
            sem_t[s])
    for h in top:
        h.wait()
    for h in bot:
        h.wait()


def _adj_body(xl_ref, xt_ref, dly_ref, w_ref, wt_ref, out_ref,
              emb_s, a_s, bb_s):
    i = pl.program_id(0)
    rows = out_ref.shape[0]

    @pl.when(i == 0)
    def _build():
        dcol = jax.lax.dot_general(
            dly_ref[...], w_ref[_DIM:_DIM + 1, :], (((0,), (0,)), ((), ())),
            precision=_HP)
        emb_s[:_N1, :] = jax.lax.dot(xl_ref[...], w_ref[:_DIM, :],
                                     precision=_HP)
        emb_s[_N1:, :] = jax.lax.dot(xt_ref[...], w_ref[:_DIM, :],
                                     precision=_HP)
        emb = emb_s[...] + dcol + wt_ref[2 * _HEADS:2 * _HEADS + 1, :]
        emb_s[...] = emb
        for h in range(_HEADS):
            ah = emb * wt_ref[h:h + 1, :]
            bh = emb * wt_ref[_HEADS + h:_HEADS + h + 1, :]
            na = jnp.sqrt(jnp.sum(ah * ah, axis=1, keepdims=True))
            nb = jnp.sqrt(jnp.sum(bh * bh, axis=1, keepdims=True))
            a_s[:, h * _EMB:(h + 1) * _EMB] = (
                ah / jnp.maximum(na, 1e-20)).astype(jnp.bfloat16)
            bb_s[:, h * _EMB:(h + 1) * _EMB] = (
                bh / jnp.maximum(nb, 1e-20)).astype(jnp.bfloat16)

    acc = jnp.zeros((rows, _N), jnp.float32)
    for h in range(_HEADS):
        ah = a_s[pl.ds(i * rows, rows), h * _EMB:(h + 1) * _EMB]
        bh = bb_s[:, h * _EMB:(h + 1) * _EMB]
        dots = jax.lax.dot_general(ah, bh, (((1,), (1,)), ((), ())),
                                   preferred_element_type=jnp.float32)
        acc = acc + jnp.tanh(dots * 0.5)
    out_ref[...] = acc * (0.5 / _HEADS) + 0.5


def kernel(lm_X, lm_Y, tg_X, tg_Y, lm_delay, tg_delay, emb_W, emb_b, w1, w2):
    dly_row = jnp.concatenate([lm_delay, tg_delay])[None, :]
    wt = jnp.concatenate([w1.T, w2.T, emb_b[None, :]], axis=0)

    k_rows = 512
    nn_idx, y_pred = pl.pallas_call(
        _knn_body,
        grid=(_N1 // k_rows,),
        in_specs=[
            pl.BlockSpec((k_rows, _DIM), lambda i: (i, 0)),
            pl.BlockSpec((_DIM, _N1), lambda i: (0, 0)),
            pl.BlockSpec((_N1, 2), lambda i: (0, 0)),
        ],
        out_specs=[
            pl.BlockSpec((k_rows, _KP), lambda i: (i, 0)),
            pl.BlockSpec((k_rows, 2), lambda i: (i, 0)),
        ],
        out_shape=[
            jax.ShapeDtypeStruct((_N1, _KP), jnp.int32),
            jax.ShapeDtypeStruct((_N1, 2), jnp.float32),
        ],
    )(lm_X, lm_X.T, lm_Y)

    sc_fn = pl.kernel(
        _sc_teacher,
        out_type=jax.ShapeDtypeStruct((_N, _N), jnp.float32),
        mesh=plsc.VectorSubcoreMesh(core_axis_name="c", subcore_axis_name="s"),
        compiler_params=pltpu.CompilerParams(needs_layout_passes=False),
        scratch_types=[
            pltpu.VMEM((_RPW, _KP), jnp.int32),
            pltpu.VMEM((2 * _NSLOT, _N), jnp.float32),
            pltpu.VMEM((2, _N), jnp.float32),
            pltpu.SemaphoreType.DMA,
        ] + [pltpu.SemaphoreType.DMA] * _NSLOT,
    )
    teacher = sc_fn(nn_idx)

    wt_dep, _ = jax.lax.optimization_barrier((wt, nn_idx))

    a_rows = 512
    adj = pl.pallas_call(
        _adj_body,
        grid=(_N // a_rows,),
        in_specs=[
            pl.BlockSpec((_N1, _DIM), lambda i: (0, 0)),
            pl.BlockSpec((_N2, _DIM), lambda i: (0, 0)),
            pl.BlockSpec((1, _N), lambda i: (0, 0)),
            pl.BlockSpec((_DIM + 1, _EMB), lambda i: (0, 0)),
            pl.BlockSpec((2 * _HEADS + 1, _EMB), lambda i: (0, 0)),
        ],
        out_specs=pl.BlockSpec((a_rows, _N), lambda i: (i, 0)),
        out_shape=jax.ShapeDtypeStruct((_N, _N), jnp.float32),
        scratch_shapes=[
            pltpu.VMEM((_N, _EMB), jnp.float32),
            pltpu.VMEM((_N, _HEADS * _EMB), jnp.bfloat16),
            pltpu.VMEM((_N, _HEADS * _EMB), jnp.bfloat16),
        ],
    )(lm_X, tg_X, dly_row, emb_W, wt_dep)

    return y_pred, adj, teacher
